# baseline (speedup 1.0000x reference)
import jax
import jax.numpy as jnp
from jax.experimental import pallas as pl
from jax.experimental.pallas import tpu as pltpu

_FG = 1
_POS_IOU = 0.2
_ENT_THR = 0.1
_RM_THR = 0.9
_THETA = 0.1

_INTERPRET = False


def _fused_body(x_ref, vmat_ref, pjg_ref, gjg_ref, phi_ref, plo_ref,
                tgt_row_ref, gts_row_ref, gts_col_ref, u_ref,
                iou_out_ref, cls_out_ref,
                maxprob_s, clsl_s, colsum_s):
    s = pl.program_id(0)
    nsteps = pl.num_programs(0)
    cc = x_ref.shape[1]
    m = vmat_ref.shape[0]
    c = vmat_ref.shape[1] - 1
    n_valid = m - (m // 100)

    maxprob_s[pl.ds(s, 1), :] = jnp.max(x_ref[...], axis=(-2, -1))

    base = s * cc
    lane_m = jax.lax.broadcasted_iota(jnp.int32, (cc, m), 1)
    pjg = pjg_ref[pl.ds(base, cc), :]
    p_onehot = (pjg == lane_m).astype(jnp.float32)
    g = jnp.dot(p_onehot, vmat_ref[...],
                preferred_element_type=jnp.float32)
    a = g[:, 0:1]
    logits = g[:, 1:]

    gjg = gjg_ref[pl.ds(base, cc), :]
    t_onehot = (gjg == lane_m).astype(jnp.float32)
    tid = jnp.sum(t_onehot * tgt_row_ref[...], axis=1, keepdims=True)
    cls = tid.astype(jnp.int32)

    mx = jnp.max(logits, axis=1, keepdims=True)
    lse = mx + jnp.log(jnp.sum(jnp.exp(logits - mx), axis=1, keepdims=True))
    lane_c = jax.lax.broadcasted_iota(jnp.int32, (cc, c), 1)
    picked = jnp.sum(jnp.where(lane_c == cls, logits, 0.0), axis=1,
                     keepdims=True)
    clsl_s[pl.ds(base, cc), :] = lse - picked

    valid_i = (jax.lax.broadcasted_iota(jnp.int32, (cc, 1), 0) + base) < n_valid
    d = jnp.abs(a - gts_row_ref[...])
    f = jnp.where(d < _THETA, d * d * (1.0 / (2.0 * _THETA)), d - 0.5 * _THETA)
    f = jnp.where(valid_i, f, 0.0)
    contrib = jnp.sum(f, axis=0, keepdims=True)

    @pl.when(s == 0)
    def _():
        colsum_s[...] = contrib

    @pl.when(s > 0)
    def _():
        colsum_s[...] += contrib

    @pl.when(s == nsteps - 1)
    def _():
        mat = maxprob_s[...]
        lane_s = jax.lax.broadcasted_iota(jnp.int32, (m, nsteps), 1)
        row_sel = (phi_ref[...] == lane_s).astype(jnp.float32)
        t1 = jnp.dot(row_sel, mat, preferred_element_type=jnp.float32)
        lane_cc = jax.lax.broadcasted_iota(jnp.int32, (m, cc), 1)
        mp = jnp.sum(jnp.where(plo_ref[...] == lane_cc, t1, 0.0), axis=1,
                     keepdims=True)
        removed = (mp < _ENT_THR) & (u_ref[...] < _RM_THR)
        w = jnp.where(removed, 0.0,
                      jnp.where(gts_col_ref[...] < _POS_IOU, 1.0, 2.0))
        valid = jax.lax.broadcasted_iota(jnp.int32, (m, 1), 0) < n_valid
        w = jnp.where(valid, w, 0.0)
        wsum = jnp.sum(w) + 0.0001
        cls_loss = jnp.sum(clsl_s[...] * w) / wsum
        iou_num = jnp.dot(colsum_s[...], w,
                          preferred_element_type=jnp.float32)
        iou_out_ref[...] = iou_num / wsum
        cls_out_ref[...] = jnp.reshape(cls_loss, (1, 1))


@jax.jit
def kernel(cls_logits, iou_scores, map_ious, pred_mask_prob, target_ids,
           map_indices):
    bs, ch, c = cls_logits.shape
    ht, wd = pred_mask_prob.shape[2], pred_mask_prob.shape[3]
    rows = bs * ch
    cc = 20
    nc = ch // cc
    nsteps = bs * nc

    k = ch - _FG
    pad = rows - bs * k
    zpad_i = jnp.zeros((pad,), jnp.int32)
    zpad_f = jnp.zeros((pad,), jnp.float32)

    pj = map_indices[:, 0, _FG:].astype(jnp.int32)
    gj = map_indices[:, 1, _FG:].astype(jnp.int32)
    off = (jnp.arange(bs, dtype=jnp.int32) * ch)[:, None]
    pjg = jnp.concatenate([(pj + off).reshape(-1), zpad_i])
    gjg = jnp.concatenate([(gj + off).reshape(-1), zpad_i])
    phi = (pjg // cc)[:, None]
    plo = (pjg % cc)[:, None]
    vmat = jnp.concatenate(
        [iou_scores.reshape(rows, 1), cls_logits.reshape(rows, c)], axis=1)
    tgt_row = target_ids.astype(jnp.float32).reshape(1, rows)
    iou = map_ious[:, _FG:].astype(jnp.float32).reshape(-1)
    iou_p = jnp.concatenate([iou, zpad_f])
    gts_row = iou_p.reshape(1, rows)
    gts_col = iou_p.reshape(rows, 1)
    u = jax.random.uniform(jax.random.key(42), (bs, k), dtype=jnp.float32)
    u_col = jnp.concatenate([u.reshape(-1), zpad_f]).reshape(rows, 1)

    full = lambda shape: pl.BlockSpec(shape, lambda s: (0,) * len(shape))
    iou_loss, cls_loss = pl.pallas_call(
        _fused_body,
        grid=(nsteps,),
        in_specs=[
            pl.BlockSpec((1, cc, ht, wd), lambda s: (s // nc, s % nc, 0, 0)),
            full((rows, 1 + c)),
            full((rows, 1)),
            full((rows, 1)),
            full((rows, 1)),
            full((rows, 1)),
            full((1, rows)),
            full((1, rows)),
            full((rows, 1)),
            full((rows, 1)),
        ],
        out_specs=[full((1, 1)), full((1, 1))],
        out_shape=[jax.ShapeDtypeStruct((1, 1), jnp.float32),
                   jax.ShapeDtypeStruct((1, 1), jnp.float32)],
        scratch_shapes=[
            pltpu.VMEM((nsteps, cc), jnp.float32),
            pltpu.VMEM((rows, 1), jnp.float32),
            pltpu.VMEM((1, rows), jnp.float32),
        ],
        interpret=_INTERPRET,
    )(pred_mask_prob, vmat, pjg[:, None], gjg[:, None], phi, plo,
      tgt_row, gts_row, gts_col, u_col)
    return (iou_loss[0, 0], cls_loss[0, 0])
